# baseline (speedup 1.0000x reference)
import functools
import jax
import jax.numpy as jnp
from jax import lax
from jax.experimental import pallas as pl
from jax.experimental.pallas import tpu as pltpu
from jax.experimental.pallas import tpu_sc as plsc

B = 2048
OBS = 1024
ACT = 256
HID = 1024
E = 8
K = 2
NP = B * K
TILE = 256
PN = NP + E * TILE
NT = PN // TILE
NTPAD = 32
VP = 128

NC = 2
NS = 16
NW = NC * NS
SLOTS_W = PN // NW
PAIRS_S = NP // NS
TOK_W = B // NW


def _top2(logits):
    ef = lax.broadcasted_iota(jnp.int32, logits.shape, 1).astype(jnp.float32)
    m1 = jnp.max(logits, axis=-1, keepdims=True)
    i1 = jnp.min(jnp.where(logits == m1, ef, float(E)), axis=-1, keepdims=True)
    mask1 = ef == i1
    l2 = jnp.where(mask1, -1e30, logits)
    m2 = jnp.max(l2, axis=-1, keepdims=True)
    i2 = jnp.min(jnp.where(l2 == m2, ef, float(E)), axis=-1, keepdims=True)
    mask2 = ef == i2
    e2 = jnp.exp(m2 - m1)
    s = 1.0 + e2
    return mask1, mask2, 1.0 / s, e2 / s


def _router_head(x16, wg_ref, pos_ref, g_ref, tile_ref, ohs, gs, rs):
    dn = (((1,), (0,)), ((), ()))
    logits = lax.dot_general(x16, wg_ref[...].astype(jnp.bfloat16), dn,
                             preferred_element_type=jnp.float32)
    mask1, mask2, g1, g2 = _top2(logits)
    ohs[0:B, :] = mask1.astype(jnp.float32)
    ohs[B:NP, :] = mask2.astype(jnp.float32)
    gs[0:B, :] = g1
    gs[B:NP, :] = g2

    ri = lax.broadcasted_iota(jnp.int32, (128, 128), 0)
    ci = lax.broadcasted_iota(jnp.int32, (128, 128), 1)
    ltri = jnp.where(ci < ri, 1.0, 0.0).astype(jnp.bfloat16)

    def body(c, carry):
        chunk = ohs[pl.ds(c * 128, 128), :]
        excl = lax.dot_general(ltri, chunk.astype(jnp.bfloat16), dn,
                               preferred_element_type=jnp.float32)
        rs[pl.ds(c * 128, 128), :] = excl + carry
        return carry + jnp.sum(chunk, axis=0, keepdims=True)

    counts = lax.fori_loop(0, NP // 128, body, jnp.zeros((1, E), jnp.float32))
    pc = jnp.floor((counts + float(TILE - 1)) * (1.0 / TILE)) * float(TILE)
    ui = lax.broadcasted_iota(jnp.int32, (E, E), 0)
    uj = lax.broadcasted_iota(jnp.int32, (E, E), 1)
    utri = jnp.where(ui < uj, 1.0, 0.0).astype(jnp.bfloat16)
    off = lax.dot_general(pc.astype(jnp.bfloat16), utri, dn,
                          preferred_element_type=jnp.float32)
    posf = jnp.sum((rs[...] + off) * ohs[...], axis=1, keepdims=True)
    pos_ref[...] = posf.astype(jnp.int32)
    g_ref[...] = gs[...]
    tif = lax.broadcasted_iota(jnp.int32, (NTPAD, E), 0).astype(jnp.float32)
    ge = jnp.where(tif * float(TILE) >= (off + pc), 1.0, 0.0)
    te = jnp.minimum(jnp.sum(ge, axis=1, keepdims=True), float(E - 1))
    tile_ref[...] = te.astype(jnp.int32)


def _router_body(x_ref, pwg_ref, vwg_ref,
                 ppos_ref, pg_ref, ptile_ref, vpos_ref, vg_ref, vtile_ref,
                 ohs, gs, rs):
    x16 = x_ref[...].astype(jnp.bfloat16)
    _router_head(x16, pwg_ref, ppos_ref, pg_ref, ptile_ref, ohs, gs, rs)
    _router_head(x16, vwg_ref, vpos_ref, vg_ref, vtile_ref, ohs, gs, rs)


def _router(x, pw_gate, vw_gate):
    out = pl.pallas_call(
        _router_body,
        grid=(1,),
        in_specs=[
            pl.BlockSpec((B, OBS), lambda i: (0, 0)),
            pl.BlockSpec((OBS, E), lambda i: (0, 0)),
            pl.BlockSpec((OBS, E), lambda i: (0, 0)),
        ],
        out_specs=[
            pl.BlockSpec((NP, 1), lambda i: (0, 0)),
            pl.BlockSpec((NP, 1), lambda i: (0, 0)),
            pl.BlockSpec((NTPAD, 1), lambda i: (0, 0)),
            pl.BlockSpec((NP, 1), lambda i: (0, 0)),
            pl.BlockSpec((NP, 1), lambda i: (0, 0)),
            pl.BlockSpec((NTPAD, 1), lambda i: (0, 0)),
        ],
        out_shape=[
            jax.ShapeDtypeStruct((NP, 1), jnp.int32),
            jax.ShapeDtypeStruct((NP, 1), jnp.float32),
            jax.ShapeDtypeStruct((NTPAD, 1), jnp.int32),
            jax.ShapeDtypeStruct((NP, 1), jnp.int32),
            jax.ShapeDtypeStruct((NP, 1), jnp.float32),
            jax.ShapeDtypeStruct((NTPAD, 1), jnp.int32),
        ],
        scratch_shapes=[
            pltpu.VMEM((NP, E), jnp.float32),
            pltpu.VMEM((NP, 1), jnp.float32),
            pltpu.VMEM((NP, E), jnp.float32),
        ],
    )(x, pw_gate, vw_gate)
    return out


def _dispatch_body(xb_hbm, ppos_hbm, pg_hbm, vpos_hbm, vg_hbm,
                   xsp_hbm, gsp_hbm, xsv_hbm, gsv_hbm,
                   dp8sh, dv8sh, gpsh, gvsh,
                   pb0, pb1, vb0, vb1, gb0, gb1, gb2, gb3, tkb,
                   i8b, v8b, wip8, wiv8, wgp, wgv, zf, zi8,
                   rowsa, rowsb, rowsc, rowsd,
                   sem_z, sem_l, sem_s, sem_w, sem_o, sem_ga, sem_gb,
                   sem_gc, sem_gd, sem_oa, sem_ob, sem_oc, sem_od):
    c = lax.axis_index("c")
    s = lax.axis_index("s")

    def zbody(j, _):
        zi8[pl.ds(j * 16, 16)] = jnp.zeros((16,), jnp.int32)
        return 0
    lax.fori_loop(0, (8 * PN // NS) // 16, zbody, 0)
    for j in range(PN // NS // 16):
        zf[pl.ds(j * 16, 16)] = jnp.zeros((16,), jnp.float32)
    zsl8 = pl.ds(s * (8 * PN // NS), 8 * PN // NS)
    zsl = pl.ds(s * (PN // NS), PN // NS)
    z0 = pltpu.async_copy(zi8, dp8sh.at[zsl8], sem_z)
    z1 = pltpu.async_copy(zi8, dv8sh.at[zsl8], sem_z)
    z2 = pltpu.async_copy(zf, gpsh.at[zsl], sem_z)
    z3 = pltpu.async_copy(zf, gvsh.at[zsl], sem_z)
    base0 = s * PAIRS_S
    l0 = pltpu.async_copy(ppos_hbm.at[pl.ds(base0, 128)], pb0, sem_l)
    l1 = pltpu.async_copy(ppos_hbm.at[pl.ds(base0 + 128, 128)], pb1, sem_l)
    l2 = pltpu.async_copy(vpos_hbm.at[pl.ds(base0, 128)], vb0, sem_l)
    l3 = pltpu.async_copy(vpos_hbm.at[pl.ds(base0 + 128, 128)], vb1, sem_l)
    l4 = pltpu.async_copy(pg_hbm.at[pl.ds(base0, 128)], gb0, sem_l)
    l5 = pltpu.async_copy(pg_hbm.at[pl.ds(base0 + 128, 128)], gb1, sem_l)
    l6 = pltpu.async_copy(vg_hbm.at[pl.ds(base0, 128)], gb2, sem_l)
    l7 = pltpu.async_copy(vg_hbm.at[pl.ds(base0 + 128, 128)], gb3, sem_l)
    for j in range(16):
        tkb[pl.ds(j * 16, 16)] = (base0 + j * 16 + lax.iota(jnp.int32, 16)) & (B - 1)
    for d in (z0, z1, z2, z3, l0, l1, l2, l3, l4, l5, l6, l7):
        d.wait()
    plsc.subcore_barrier()

    sg = []
    sg.append(pltpu.async_copy(gb0, gpsh.at[pb0], sem_s))
    sg.append(pltpu.async_copy(gb1, gpsh.at[pb1], sem_s))
    sg.append(pltpu.async_copy(gb2, gvsh.at[vb0], sem_s))
    sg.append(pltpu.async_copy(gb3, gvsh.at[vb1], sem_s))
    for (pb, dsh, tof) in ((pb0, dp8sh, 0), (pb1, dp8sh, 128),
                           (vb0, dv8sh, 0), (vb1, dv8sh, 128)):
        def cbody(j, _):
            pv = pb[pl.ds(j * 16, 16)]
            tv = tkb[pl.ds(tof + j * 16, 16)]
            for k in range(8):
                i8b[k][pl.ds(j * 16, 16)] = pv * 8 + k
                v8b[k][pl.ds(j * 16, 16)] = tv * 8 + k
            return 0
        lax.fori_loop(0, 8, cbody, 0)
        for k in range(8):
            sg.append(pltpu.async_copy(v8b[k], dsh.at[i8b[k]], sem_s))
        for d in sg:
            d.wait()
        sg = []
    plsc.subcore_barrier()

    base = c * (PN // NC) + s * SLOTS_W
    w0 = pltpu.async_copy(dp8sh.at[pl.ds(base * 8, SLOTS_W * 8)], wip8, sem_w)
    w2 = pltpu.async_copy(dv8sh.at[pl.ds(base * 8, SLOTS_W * 8)], wiv8, sem_w)
    w4 = pltpu.async_copy(gpsh.at[pl.ds(base, SLOTS_W)], wgp, sem_w)
    w5 = pltpu.async_copy(gvsh.at[pl.ds(base, SLOTS_W)], wgv, sem_w)
    for d in (w0, w2, w4, w5):
        d.wait()
    o0 = pltpu.async_copy(wgp, gsp_hbm.at[pl.ds(base, SLOTS_W)], sem_o)
    o1 = pltpu.async_copy(wgv, gsv_hbm.at[pl.ds(base, SLOTS_W)], sem_o)
    CH8 = 128
    NCH = SLOTS_W * 8 // CH8
    bufs = (rowsa, rowsb, rowsc, rowsd)
    sems = (sem_ga, sem_gb, sem_gc, sem_gd)
    osems = (sem_oa, sem_ob, sem_oc, sem_od)
    pend_g = [None] * 4
    pend_w = [None] * 4
    work = []
    for h in range(2):
        for j in range(NCH):
            work.append((h, j * CH8))
    for k, (h, off) in enumerate(work):
        b = k % 4
        if pend_w[b] is not None:
            pend_w[b].wait()
        idx = wip8 if h == 0 else wiv8
        pend_g[b] = pltpu.async_copy(
            xb_hbm.at[idx.at[pl.ds(off, CH8)]], bufs[b], sems[b])
        pend_g[b].wait()
        dst = xsp_hbm if h == 0 else xsv_hbm
        pend_w[b] = pltpu.async_copy(
            bufs[b], dst.at[pl.ds(base * 8 + off, CH8)], osems[b])
    for b in range(4):
        if pend_w[b] is not None:
            pend_w[b].wait()
    o0.wait()
    o1.wait()


def _dispatch(x8, ppos, pg, vpos, vg):
    mesh = plsc.VectorSubcoreMesh(core_axis_name="c", subcore_axis_name="s")
    f = pl.kernel(
        _dispatch_body,
        out_type=[
            jax.ShapeDtypeStruct((PN * 8, 128), jnp.float32),
            jax.ShapeDtypeStruct((PN,), jnp.float32),
            jax.ShapeDtypeStruct((PN * 8, 128), jnp.float32),
            jax.ShapeDtypeStruct((PN,), jnp.float32),
        ],
        mesh=mesh,
        scratch_types=[
            pltpu.VMEM_SHARED((PN * 8,), jnp.int32),
            pltpu.VMEM_SHARED((PN * 8,), jnp.int32),
            pltpu.VMEM_SHARED((PN,), jnp.float32),
            pltpu.VMEM_SHARED((PN,), jnp.float32),
            pltpu.VMEM((128,), jnp.int32),
            pltpu.VMEM((128,), jnp.int32),
            pltpu.VMEM((128,), jnp.int32),
            pltpu.VMEM((128,), jnp.int32),
            pltpu.VMEM((128,), jnp.float32),
            pltpu.VMEM((128,), jnp.float32),
            pltpu.VMEM((128,), jnp.float32),
            pltpu.VMEM((128,), jnp.float32),
            pltpu.VMEM((PAIRS_S,), jnp.int32),
            [pltpu.VMEM((128,), jnp.int32)] * 8,
            [pltpu.VMEM((128,), jnp.int32)] * 8,
            pltpu.VMEM((SLOTS_W * 8,), jnp.int32),
            pltpu.VMEM((SLOTS_W * 8,), jnp.int32),
            pltpu.VMEM((SLOTS_W,), jnp.float32),
            pltpu.VMEM((SLOTS_W,), jnp.float32),
            pltpu.VMEM((PN // NS,), jnp.float32),
            pltpu.VMEM((8 * PN // NS,), jnp.int32),
            pltpu.VMEM((128, 128), jnp.float32),
            pltpu.VMEM((128, 128), jnp.float32),
            pltpu.VMEM((128, 128), jnp.float32),
            pltpu.VMEM((128, 128), jnp.float32),
        ] + [pltpu.SemaphoreType.DMA] * 13,
    )
    return f(x8, ppos, pg, vpos, vg)


def _mm_body(tile_ref, xs_ref, w1_ref, b1_ref, w2_ref, b2_ref, gs_ref, out_ref):
    x16 = xs_ref[...].astype(jnp.bfloat16)
    w1 = w1_ref[0].astype(jnp.bfloat16)
    h = jnp.dot(x16, w1, preferred_element_type=jnp.float32)
    h = jnp.maximum(h + b1_ref[0], 0.0).astype(jnp.bfloat16)
    w2 = w2_ref[0].astype(jnp.bfloat16)
    y = jnp.dot(h, w2, preferred_element_type=jnp.float32) + b2_ref[0]
    out_ref[...] = y * gs_ref[...]


def _grouped_mm(xs2, gs2, tile_map, W1, b1, W2, b2, act):
    grid_spec = pltpu.PrefetchScalarGridSpec(
        num_scalar_prefetch=1,
        grid=(NT,),
        in_specs=[
            pl.BlockSpec((TILE, OBS), lambda t, tm: (t, 0)),
            pl.BlockSpec((1, OBS, HID), lambda t, tm: (tm[t], 0, 0)),
            pl.BlockSpec((1, 1, HID), lambda t, tm: (tm[t], 0, 0)),
            pl.BlockSpec((1, HID, act), lambda t, tm: (tm[t], 0, 0)),
            pl.BlockSpec((1, 1, act), lambda t, tm: (tm[t], 0, 0)),
            pl.BlockSpec((TILE, 1), lambda t, tm: (t, 0)),
        ],
        out_specs=pl.BlockSpec((TILE, act), lambda t, tm: (t, 0)),
    )
    return pl.pallas_call(
        _mm_body,
        grid_spec=grid_spec,
        out_shape=jax.ShapeDtypeStruct((PN, act), jnp.float32),
        compiler_params=pltpu.CompilerParams(
            dimension_semantics=("arbitrary",),
        ),
    )(tile_map, xs2, W1, b1.reshape(E, 1, HID), W2, b2.reshape(E, 1, act), gs2)


def _combine_body(ysp_hbm, ysv_hbm, ppos_hbm, vpos_hbm, outp_hbm, outv_hbm,
                  pb0, pb1, vb0, vb1, i00, i01, i10, i11,
                  bufa, bufb, bufv, sem_l, sem_g, sem_w):
    c = lax.axis_index("c")
    s = lax.axis_index("s")
    wid = s * NC + c
    base = wid * TOK_W
    l0 = pltpu.async_copy(ppos_hbm.at[pl.ds(base, TOK_W)], pb0, sem_l)
    l1 = pltpu.async_copy(ppos_hbm.at[pl.ds(B + base, TOK_W)], pb1, sem_l)
    l2 = pltpu.async_copy(vpos_hbm.at[pl.ds(base, TOK_W)], vb0, sem_l)
    l3 = pltpu.async_copy(vpos_hbm.at[pl.ds(B + base, TOK_W)], vb1, sem_l)
    for d in (l0, l1, l2, l3):
        d.wait()
    for j in range(TOK_W // 16):
        v0 = pb0[pl.ds(j * 16, 16)]
        i00[pl.ds(j * 16, 16)] = v0 + v0
        i01[pl.ds(j * 16, 16)] = v0 + v0 + 1
        v1 = pb1[pl.ds(j * 16, 16)]
        i10[pl.ds(j * 16, 16)] = v1 + v1
        i11[pl.ds(j * 16, 16)] = v1 + v1 + 1
    g0 = pltpu.async_copy(ysp_hbm.at[i00], bufa, sem_g)
    g1 = pltpu.async_copy(ysp_hbm.at[i01], bufb, sem_g)
    g2 = pltpu.async_copy(ysv_hbm.at[vb0], bufv, sem_g)
    for d in (g0, g1, g2):
        d.wait()
    a0 = pltpu.async_copy(ysp_hbm.at[i10], bufa, sem_g, add=True)
    a1 = pltpu.async_copy(ysp_hbm.at[i11], bufb, sem_g, add=True)
    a2 = pltpu.async_copy(ysv_hbm.at[vb1], bufv, sem_g, add=True)
    for d in (a0, a1, a2):
        d.wait()
    w0 = pltpu.async_copy(bufa, outp_hbm.at[pl.ds(base, TOK_W), 0], sem_w)
    w1 = pltpu.async_copy(bufb, outp_hbm.at[pl.ds(base, TOK_W), 1], sem_w)
    w2 = pltpu.async_copy(bufv, outv_hbm.at[pl.ds(base, TOK_W)], sem_w)
    for d in (w0, w1, w2):
        d.wait()


def _combine(ysp2, ysv, ppos, vpos):
    mesh = plsc.VectorSubcoreMesh(core_axis_name="c", subcore_axis_name="s")
    f = pl.kernel(
        _combine_body,
        out_type=[
            jax.ShapeDtypeStruct((B, ACT // 128, 128), jnp.float32),
            jax.ShapeDtypeStruct((B, VP), jnp.float32),
        ],
        mesh=mesh,
        scratch_types=[
            pltpu.VMEM((TOK_W,), jnp.int32),
            pltpu.VMEM((TOK_W,), jnp.int32),
            pltpu.VMEM((TOK_W,), jnp.int32),
            pltpu.VMEM((TOK_W,), jnp.int32),
            pltpu.VMEM((TOK_W,), jnp.int32),
            pltpu.VMEM((TOK_W,), jnp.int32),
            pltpu.VMEM((TOK_W,), jnp.int32),
            pltpu.VMEM((TOK_W,), jnp.int32),
            pltpu.VMEM((TOK_W, 128), jnp.float32),
            pltpu.VMEM((TOK_W, 128), jnp.float32),
            pltpu.VMEM((TOK_W, VP), jnp.float32),
            pltpu.SemaphoreType.DMA,
            pltpu.SemaphoreType.DMA,
            pltpu.SemaphoreType.DMA,
        ],
    )
    return f(ysp2, ysv, ppos, vpos)


def kernel(x, pw_gate, pW1, pb1, pW2, pb2, vw_gate, vW1, vb1, vW2, vb2):
    ppos, pg, ptile, vpos, vg, vtile = _router(x, pw_gate, vw_gate)
    ppos1 = ppos.reshape(NP)
    vpos1 = vpos.reshape(NP)
    xsp8, gsp, xsv8, gsv = _dispatch(x.reshape(B * 8, 128), ppos1,
                                     pg.reshape(NP), vpos1, vg.reshape(NP))
    xsp = xsp8.reshape(PN, OBS)
    xsv = xsv8.reshape(PN, OBS)
    ptm = ptile.reshape(NTPAD)[:NT]
    vtm = vtile.reshape(NTPAD)[:NT]
    ysp = _grouped_mm(xsp, gsp.reshape(PN, 1), ptm, pW1, pb1, pW2, pb2, ACT)
    vW2p = jnp.pad(vW2, ((0, 0), (0, 0), (0, VP - 1)))
    vb2p = jnp.pad(vb2, ((0, 0), (0, VP - 1)))
    ysv = _grouped_mm(xsv, gsv.reshape(PN, 1), vtm, vW1, vb1, vW2p, vb2p, VP)
    outp3, outv = _combine(ysp.reshape(PN * 2, 128), ysv, ppos1, vpos1)
    return (outp3.reshape(B, ACT), outv[:, 0])


# speedup vs baseline: 4.1556x; 4.1556x over previous
import functools
import jax
import jax.numpy as jnp
from jax import lax
from jax.experimental import pallas as pl
from jax.experimental.pallas import tpu as pltpu
from jax.experimental.pallas import tpu_sc as plsc

B = 2048
OBS = 1024
ACT = 256
HID = 1024
E = 8
K = 2
NP = B * K
TILE = 256
PN = NP + E * TILE
NT = PN // TILE
NTPAD = 32
VP = 128

NC = 2
NS = 16
NW = NC * NS
SLOTS_W = PN // NW
PAIRS_S = NP // NS
TOK_W = B // NW


def _top2(logits):
    ef = lax.broadcasted_iota(jnp.int32, logits.shape, 1).astype(jnp.float32)
    m1 = jnp.max(logits, axis=-1, keepdims=True)
    i1 = jnp.min(jnp.where(logits == m1, ef, float(E)), axis=-1, keepdims=True)
    mask1 = ef == i1
    l2 = jnp.where(mask1, -1e30, logits)
    m2 = jnp.max(l2, axis=-1, keepdims=True)
    i2 = jnp.min(jnp.where(l2 == m2, ef, float(E)), axis=-1, keepdims=True)
    mask2 = ef == i2
    e2 = jnp.exp(m2 - m1)
    s = 1.0 + e2
    return mask1, mask2, 1.0 / s, e2 / s


def _router_head(x16, wg_ref, pos_ref, g_ref, tile_ref, ohs, gs, rs):
    dn = (((1,), (0,)), ((), ()))
    logits = lax.dot_general(x16, wg_ref[...].astype(jnp.bfloat16), dn,
                             preferred_element_type=jnp.float32)
    mask1, mask2, g1, g2 = _top2(logits)
    ohs[0:B, :] = mask1.astype(jnp.float32)
    ohs[B:NP, :] = mask2.astype(jnp.float32)
    gs[0:B, :] = g1
    gs[B:NP, :] = g2

    ri = lax.broadcasted_iota(jnp.int32, (128, 128), 0)
    ci = lax.broadcasted_iota(jnp.int32, (128, 128), 1)
    ltri = jnp.where(ci < ri, 1.0, 0.0).astype(jnp.bfloat16)

    def body(c, carry):
        chunk = ohs[pl.ds(c * 128, 128), :]
        excl = lax.dot_general(ltri, chunk.astype(jnp.bfloat16), dn,
                               preferred_element_type=jnp.float32)
        rs[pl.ds(c * 128, 128), :] = excl + carry
        return carry + jnp.sum(chunk, axis=0, keepdims=True)

    counts = lax.fori_loop(0, NP // 128, body, jnp.zeros((1, E), jnp.float32))
    pc = jnp.floor((counts + float(TILE - 1)) * (1.0 / TILE)) * float(TILE)
    ui = lax.broadcasted_iota(jnp.int32, (E, E), 0)
    uj = lax.broadcasted_iota(jnp.int32, (E, E), 1)
    utri = jnp.where(ui < uj, 1.0, 0.0).astype(jnp.bfloat16)
    off = lax.dot_general(pc.astype(jnp.bfloat16), utri, dn,
                          preferred_element_type=jnp.float32)
    posf = jnp.sum((rs[...] + off) * ohs[...], axis=1, keepdims=True)
    pos_ref[...] = posf.astype(jnp.int32)
    g_ref[...] = gs[...]
    tif = lax.broadcasted_iota(jnp.int32, (NTPAD, E), 0).astype(jnp.float32)
    ge = jnp.where(tif * float(TILE) >= (off + pc), 1.0, 0.0)
    te = jnp.minimum(jnp.sum(ge, axis=1, keepdims=True), float(E - 1))
    tile_ref[...] = te.astype(jnp.int32)


def _router_body(x_ref, pwg_ref, vwg_ref,
                 ppos_ref, pg_ref, ptile_ref, vpos_ref, vg_ref, vtile_ref,
                 ohs, gs, rs):
    x16 = x_ref[...].astype(jnp.bfloat16)
    _router_head(x16, pwg_ref, ppos_ref, pg_ref, ptile_ref, ohs, gs, rs)
    _router_head(x16, vwg_ref, vpos_ref, vg_ref, vtile_ref, ohs, gs, rs)


def _router(x, pw_gate, vw_gate):
    out = pl.pallas_call(
        _router_body,
        grid=(1,),
        in_specs=[
            pl.BlockSpec((B, OBS), lambda i: (0, 0)),
            pl.BlockSpec((OBS, E), lambda i: (0, 0)),
            pl.BlockSpec((OBS, E), lambda i: (0, 0)),
        ],
        out_specs=[
            pl.BlockSpec((NP, 1), lambda i: (0, 0)),
            pl.BlockSpec((NP, 1), lambda i: (0, 0)),
            pl.BlockSpec((NTPAD, 1), lambda i: (0, 0)),
            pl.BlockSpec((NP, 1), lambda i: (0, 0)),
            pl.BlockSpec((NP, 1), lambda i: (0, 0)),
            pl.BlockSpec((NTPAD, 1), lambda i: (0, 0)),
        ],
        out_shape=[
            jax.ShapeDtypeStruct((NP, 1), jnp.int32),
            jax.ShapeDtypeStruct((NP, 1), jnp.float32),
            jax.ShapeDtypeStruct((NTPAD, 1), jnp.int32),
            jax.ShapeDtypeStruct((NP, 1), jnp.int32),
            jax.ShapeDtypeStruct((NP, 1), jnp.float32),
            jax.ShapeDtypeStruct((NTPAD, 1), jnp.int32),
        ],
        scratch_shapes=[
            pltpu.VMEM((NP, E), jnp.float32),
            pltpu.VMEM((NP, 1), jnp.float32),
            pltpu.VMEM((NP, E), jnp.float32),
        ],
    )(x, pw_gate, vw_gate)
    return out


def _dispatch_body(xb_hbm, ppos_hbm, pg_hbm, vpos_hbm, vg_hbm,
                   xsp_hbm, gsp_hbm, xsv_hbm, gsv_hbm,
                   disp, gsb, posb, gb, rowsa, rowsb,
                   sem_l, sem_ga, sem_gb, sem_oa, sem_ob, sem_o):
    c = lax.axis_index("c")
    s = lax.axis_index("s")
    base = c * (PN // NC) + s * SLOTS_W
    lane = lax.iota(jnp.int32, 16)

    def zbody(j, _):
        disp[pl.ds(j * 16, 16)] = jnp.zeros((16,), jnp.int32)
        return 0
    l0 = pltpu.async_copy(ppos_hbm, posb, sem_l)
    l1 = pltpu.async_copy(pg_hbm, gb, sem_l)
    lax.fori_loop(0, PN // 16, zbody, 0)

    CH = SLOTS_W // 4
    for h in range(2):
        l0.wait()
        l1.wait()

        def sbody(i, _):
            pv = posb[pl.ds(i * 16, 16)]
            gv = gb[pl.ds(i * 16, 16)]
            tok = (i * 16 + lane) & (B - 1)
            plsc.store_scatter(disp, [pv], tok)
            plsc.store_scatter(gsb, [pv], gv)
            return 0
        lax.fori_loop(0, NP // 16, sbody, 0)
        if h == 0:
            l0 = pltpu.async_copy(vpos_hbm, posb, sem_l)
            l1 = pltpu.async_copy(vg_hbm, gb, sem_l)
        dst = xsp_hbm if h == 0 else xsv_hbm
        gdst = gsp_hbm if h == 0 else gsv_hbm
        og = pltpu.async_copy(gsb.at[pl.ds(base, SLOTS_W)],
                              gdst.at[pl.ds(base, SLOTS_W)], sem_o)
        bufs = (rowsa, rowsb)
        sems = (sem_ga, sem_gb)
        osems = (sem_oa, sem_ob)
        pend_w = [None, None]
        for j in range(4):
            b = j % 2
            if pend_w[b] is not None:
                pend_w[b].wait()
            g = pltpu.async_copy(
                xb_hbm.at[disp.at[pl.ds(base + j * CH, CH)]], bufs[b], sems[b])
            g.wait()
            pend_w[b] = pltpu.async_copy(
                bufs[b], dst.at[pl.ds(base + j * CH, CH)], osems[b])
        for b in range(2):
            pend_w[b].wait()
        og.wait()


def _dispatch(x, ppos, pg, vpos, vg):
    mesh = plsc.VectorSubcoreMesh(core_axis_name="c", subcore_axis_name="s")
    f = pl.kernel(
        _dispatch_body,
        out_type=[
            jax.ShapeDtypeStruct((PN, OBS), jnp.float32),
            jax.ShapeDtypeStruct((PN,), jnp.float32),
            jax.ShapeDtypeStruct((PN, OBS), jnp.float32),
            jax.ShapeDtypeStruct((PN,), jnp.float32),
        ],
        mesh=mesh,
        scratch_types=[
            pltpu.VMEM((PN,), jnp.int32),
            pltpu.VMEM((PN,), jnp.float32),
            pltpu.VMEM((NP,), jnp.int32),
            pltpu.VMEM((NP,), jnp.float32),
            pltpu.VMEM((SLOTS_W // 4, OBS), jnp.float32),
            pltpu.VMEM((SLOTS_W // 4, OBS), jnp.float32),
        ] + [pltpu.SemaphoreType.DMA] * 6,
        compiler_params=pltpu.CompilerParams(needs_layout_passes=False),
    )
    return f(x, ppos, pg, vpos, vg)


def _mm_body(tile_ref, xs_ref, w1_ref, b1_ref, w2_ref, b2_ref, gs_ref, out_ref):
    x16 = xs_ref[...].astype(jnp.bfloat16)
    w1 = w1_ref[0].astype(jnp.bfloat16)
    h = jnp.dot(x16, w1, preferred_element_type=jnp.float32)
    h = jnp.maximum(h + b1_ref[0], 0.0).astype(jnp.bfloat16)
    w2 = w2_ref[0].astype(jnp.bfloat16)
    y = jnp.dot(h, w2, preferred_element_type=jnp.float32) + b2_ref[0]
    out_ref[...] = y * gs_ref[...]


def _grouped_mm(xs2, gs2, tile_map, W1, b1, W2, b2, act):
    grid_spec = pltpu.PrefetchScalarGridSpec(
        num_scalar_prefetch=1,
        grid=(NT,),
        in_specs=[
            pl.BlockSpec((TILE, OBS), lambda t, tm: (t, 0)),
            pl.BlockSpec((1, OBS, HID), lambda t, tm: (tm[t], 0, 0)),
            pl.BlockSpec((1, 1, HID), lambda t, tm: (tm[t], 0, 0)),
            pl.BlockSpec((1, HID, act), lambda t, tm: (tm[t], 0, 0)),
            pl.BlockSpec((1, 1, act), lambda t, tm: (tm[t], 0, 0)),
            pl.BlockSpec((TILE, 1), lambda t, tm: (t, 0)),
        ],
        out_specs=pl.BlockSpec((TILE, act), lambda t, tm: (t, 0)),
    )
    return pl.pallas_call(
        _mm_body,
        grid_spec=grid_spec,
        out_shape=jax.ShapeDtypeStruct((PN, act), jnp.float32),
        compiler_params=pltpu.CompilerParams(
            dimension_semantics=("arbitrary",),
        ),
    )(tile_map, xs2, W1, b1.reshape(E, 1, HID), W2, b2.reshape(E, 1, act), gs2)


def _combine_body(ysp_hbm, ysv_hbm, ppos_hbm, vpos_hbm, outp_hbm, outv_hbm,
                  pb0, pb1, vb0, vb1, i00, i01, i10, i11,
                  bufa, bufb, bufv, sem_l, sem_g, sem_w):
    c = lax.axis_index("c")
    s = lax.axis_index("s")
    wid = s * NC + c
    base = wid * TOK_W
    l0 = pltpu.async_copy(ppos_hbm.at[pl.ds(base, TOK_W)], pb0, sem_l)
    l1 = pltpu.async_copy(ppos_hbm.at[pl.ds(B + base, TOK_W)], pb1, sem_l)
    l2 = pltpu.async_copy(vpos_hbm.at[pl.ds(base, TOK_W)], vb0, sem_l)
    l3 = pltpu.async_copy(vpos_hbm.at[pl.ds(B + base, TOK_W)], vb1, sem_l)
    for d in (l0, l1, l2, l3):
        d.wait()
    for j in range(TOK_W // 16):
        v0 = pb0[pl.ds(j * 16, 16)]
        i00[pl.ds(j * 16, 16)] = v0 + v0
        i01[pl.ds(j * 16, 16)] = v0 + v0 + 1
        v1 = pb1[pl.ds(j * 16, 16)]
        i10[pl.ds(j * 16, 16)] = v1 + v1
        i11[pl.ds(j * 16, 16)] = v1 + v1 + 1
    g0 = pltpu.async_copy(ysp_hbm.at[i00], bufa, sem_g)
    g1 = pltpu.async_copy(ysp_hbm.at[i01], bufb, sem_g)
    g2 = pltpu.async_copy(ysv_hbm.at[vb0], bufv, sem_g)
    for d in (g0, g1, g2):
        d.wait()
    a0 = pltpu.async_copy(ysp_hbm.at[i10], bufa, sem_g, add=True)
    a1 = pltpu.async_copy(ysp_hbm.at[i11], bufb, sem_g, add=True)
    a2 = pltpu.async_copy(ysv_hbm.at[vb1], bufv, sem_g, add=True)
    for d in (a0, a1, a2):
        d.wait()
    w0 = pltpu.async_copy(bufa, outp_hbm.at[pl.ds(base, TOK_W), 0], sem_w)
    w1 = pltpu.async_copy(bufb, outp_hbm.at[pl.ds(base, TOK_W), 1], sem_w)
    w2 = pltpu.async_copy(bufv, outv_hbm.at[pl.ds(base, TOK_W)], sem_w)
    for d in (w0, w1, w2):
        d.wait()


def _combine(ysp2, ysv, ppos, vpos):
    mesh = plsc.VectorSubcoreMesh(core_axis_name="c", subcore_axis_name="s")
    f = pl.kernel(
        _combine_body,
        out_type=[
            jax.ShapeDtypeStruct((B, ACT // 128, 128), jnp.float32),
            jax.ShapeDtypeStruct((B, VP), jnp.float32),
        ],
        mesh=mesh,
        scratch_types=[
            pltpu.VMEM((TOK_W,), jnp.int32),
            pltpu.VMEM((TOK_W,), jnp.int32),
            pltpu.VMEM((TOK_W,), jnp.int32),
            pltpu.VMEM((TOK_W,), jnp.int32),
            pltpu.VMEM((TOK_W,), jnp.int32),
            pltpu.VMEM((TOK_W,), jnp.int32),
            pltpu.VMEM((TOK_W,), jnp.int32),
            pltpu.VMEM((TOK_W,), jnp.int32),
            pltpu.VMEM((TOK_W, 128), jnp.float32),
            pltpu.VMEM((TOK_W, 128), jnp.float32),
            pltpu.VMEM((TOK_W, VP), jnp.float32),
            pltpu.SemaphoreType.DMA,
            pltpu.SemaphoreType.DMA,
            pltpu.SemaphoreType.DMA,
        ],
    )
    return f(ysp2, ysv, ppos, vpos)


def kernel(x, pw_gate, pW1, pb1, pW2, pb2, vw_gate, vW1, vb1, vW2, vb2):
    ppos, pg, ptile, vpos, vg, vtile = _router(x, pw_gate, vw_gate)
    ppos1 = ppos.reshape(NP)
    vpos1 = vpos.reshape(NP)
    xsp, gsp, xsv, gsv = _dispatch(x, ppos1, pg.reshape(NP),
                                   vpos1, vg.reshape(NP))
    ptm = ptile.reshape(NTPAD)[:NT]
    vtm = vtile.reshape(NTPAD)[:NT]
    ysp = _grouped_mm(xsp, gsp.reshape(PN, 1), ptm, pW1, pb1, pW2, pb2, ACT)
    vW2p = jnp.pad(vW2, ((0, 0), (0, 0), (0, VP - 1)))
    vb2p = jnp.pad(vb2, ((0, 0), (0, VP - 1)))
    ysv = _grouped_mm(xsv, gsv.reshape(PN, 1), vtm, vW1, vb1, vW2p, vb2p, VP)
    outp3, outv = _combine(ysp.reshape(PN * 2, 128), ysv, ppos1, vpos1)
    return (outp3.reshape(B, ACT), outv[:, 0])
